# baseline (speedup 1.0000x reference)
import functools
import jax
import jax.numpy as jnp
from jax import lax
from jax.experimental import pallas as pl
from jax.experimental.pallas import tpu as pltpu
from jax.experimental.pallas import tpu_sc as plsc

NB = 256
CP = 32


def _knn_kernel(x1t_ref, x2t_ref, idx_ref, w_ref):
    b = pl.program_id(0)
    x1 = x1t_ref[0]
    x2 = x2t_ref[0]
    M = x2.shape[1]
    cross = jax.lax.dot_general(x2, x1, (((0,), (0,)), ((), ())),
                                preferred_element_type=jnp.float32)
    n1 = jnp.sum(x1 * x1, axis=0)[None, :]
    n2 = jnp.sum(x2 * x2, axis=0)[:, None]
    d = (-2.0 * cross + n1) + n2
    iota0 = jax.lax.broadcasted_iota(jnp.int32, d.shape, 0)

    firsts = []
    ws = []
    for _ in range(3):
        mval = jnp.min(d, axis=0, keepdims=True)
        first = jnp.min(jnp.where(d == mval, iota0, M), axis=0,
                        keepdims=True)
        ws.append(1.0 / (mval + 1e-8))
        firsts.append(first + b * M)
        sel = iota0 == first
        d = jnp.where(sel, jnp.inf, d)
    norm = ws[0] + ws[1] + ws[2]
    idx_ref[0] = jnp.concatenate(firsts, axis=1)
    w_ref[0] = jnp.concatenate([w / norm for w in ws], axis=1)


def _interp_sc_kernel(idx_hbm, w_hbm, table_hbm, out_hbm, h0d_hbm, h1d_hbm,
                      ija_v, ijb_v, wja_v, wjb_v,
                      ra0_v, ra1_v, ra2_v, rb0_v, rb1_v, rb2_v,
                      out_v, sema, semb, semj):
    info = plsc.get_sparse_core_info()
    NC, NS, L = info.num_cores, info.num_subcores, info.num_lanes
    NW = NC * NS
    wid = lax.axis_index("s") * NC + lax.axis_index("c")
    total = out_hbm.shape[0]
    per_tile = total // NW
    n_chunks = per_tile // CP
    cpj = NB // CP
    n_jblk = per_tile // NB
    tile_base = wid * per_tile
    rbufs = ((ra0_v, ra1_v, ra2_v, sema), (rb0_v, rb1_v, rb2_v, semb))
    jbufs = (ija_v, wja_v), (ijb_v, wjb_v)

    def start_jblk(jb, jp):
        iv, wv = jbufs[jp]
        j = (tile_base + jb * NB) // NB
        base = pl.multiple_of(j * (3 * NB), 64)
        pltpu.async_copy(idx_hbm.at[pl.ds(base, 3 * NB)], iv, semj)
        pltpu.async_copy(w_hbm.at[pl.ds(base, 3 * NB), :], wv, semj)

    def start_chunk(off, rp, jpo):
        iv = jbufs[jpo][0]
        rows = rbufs[rp]
        for k in range(3):
            pltpu.async_copy(
                table_hbm.at[iv.at[pl.ds(k * NB + off, CP)]], rows[k],
                rows[3])

    def finish_chunk(jb, off, rp, jpo):
        iv, wv = jbufs[jpo]
        p0 = tile_base + jb * NB + off
        rows = rbufs[rp]
        for k in range(3):
            pltpu.make_async_copy(
                table_hbm.at[iv.at[pl.ds(k * NB + off, CP)]], rows[k],
                rows[3]).wait()

        def point_body(pt, _):
            v0 = wv[off + pt, :]
            v1 = wv[NB + off + pt, :]
            v2 = wv[2 * NB + off + pt, :]
            for s in range(out_v.shape[1] // L):
                sl = pl.ds(s * L, L)
                out_v[pt, sl] = (v0 * rows[0][pt, sl] + v1 * rows[1][pt, sl]
                                 + v2 * rows[2][pt, sl])
            return 0

        lax.fori_loop(0, CP, point_body, 0)
        pltpu.sync_copy(out_v, out_hbm.at[pl.ds(pl.multiple_of(p0, CP), CP)])

    def wait_jblk(jp):
        iv, wv = jbufs[jp]
        pltpu.make_async_copy(idx_hbm.at[pl.ds(0, 3 * NB)], iv, semj).wait()
        pltpu.make_async_copy(w_hbm.at[pl.ds(0, 3 * NB), :], wv, semj).wait()

    start_jblk(0, 0)
    wait_jblk(0)
    start_chunk(0, 0, 0)

    def jpair_body(t, _):
        for half in range(2):
            jb = 2 * t + half
            jp = half

            @pl.when(jb + 1 < n_jblk)
            def _():
                start_jblk(jb + 1, jp ^ 1)

            for cc in range(cpj):
                if cc < cpj - 1:
                    start_chunk((cc + 1) * CP, (cc + 1) % 2, jp)
                else:
                    @pl.when(jb + 1 < n_jblk)
                    def _():
                        wait_jblk(jp ^ 1)
                        start_chunk(0, 0, jp ^ 1)

                finish_chunk(jb, cc * CP, cc % 2, jp)
        return 0

    lax.fori_loop(0, n_jblk // 2, jpair_body, 0)


def _mlp_fused_kernel(count_inv, p1_ref, it_ref, w0_ref, b0_ref, g0_ref,
                      be0_ref, w1_ref, b1_ref, g1_ref, be1_ref,
                      h0in_ref, h1in_ref,
                      h0_ref, h1_ref, out_ref,
                      s0_ref, q0_ref, s1_ref, q1_ref):
    ph = pl.program_id(0)
    b = pl.program_id(1)
    i = pl.program_id(2)
    first = jnp.logical_and(b == 0, i == 0)

    @pl.when(ph == 0)
    def _():
        f = jnp.concatenate([p1_ref[0], it_ref[0]], axis=1)
        h = jax.lax.dot_general(f, w0_ref[...], (((1,), (1,)), ((), ())),
                                preferred_element_type=jnp.float32)
        h = h + b0_ref[...]
        h0_ref[0] = h

        @pl.when(first)
        def _():
            s0_ref[...] = jnp.zeros_like(s0_ref)
            q0_ref[...] = jnp.zeros_like(q0_ref)

        s0_ref[...] += jnp.sum(h, axis=0, keepdims=True)
        q0_ref[...] += jnp.sum(h * h, axis=0, keepdims=True)

    @pl.when(ph == 1)
    def _():
        mean = s0_ref[...] * count_inv
        var = q0_ref[...] * count_inv - mean * mean
        inv = jax.lax.rsqrt(var + 1e-5)
        scale = g0_ref[...] * inv
        shift = be0_ref[...] - mean * scale
        y = jnp.maximum(h0in_ref[0] * scale + shift, 0.0)
        h = jax.lax.dot_general(y, w1_ref[...], (((1,), (1,)), ((), ())),
                                preferred_element_type=jnp.float32)
        h = h + b1_ref[...]
        h1_ref[0] = h

        @pl.when(first)
        def _():
            s1_ref[...] = jnp.zeros_like(s1_ref)
            q1_ref[...] = jnp.zeros_like(q1_ref)

        s1_ref[...] += jnp.sum(h, axis=0, keepdims=True)
        q1_ref[...] += jnp.sum(h * h, axis=0, keepdims=True)

    @pl.when(ph == 2)
    def _():
        mean = s1_ref[...] * count_inv
        var = q1_ref[...] * count_inv - mean * mean
        inv = jax.lax.rsqrt(var + 1e-5)
        scale = g1_ref[...] * inv
        shift = be1_ref[...] - mean * scale
        out_ref[0] = jnp.maximum(h1in_ref[0] * scale + shift, 0.0)


@jax.jit
def kernel(xyz1, xyz2, points1, points2, W0, b0, g0, be0, W1, b1, g1, be1):
    B, N, _ = xyz1.shape
    M = xyz2.shape[1]
    C1 = points1.shape[-1]
    C2 = points2.shape[-1]
    CH0 = W0.shape[0]
    CH1 = W1.shape[0]
    x1t = jnp.transpose(xyz1, (0, 2, 1))
    x2t = jnp.transpose(xyz2, (0, 2, 1))
    count_inv = 1.0 / float(B * N)
    nblk = N // NB
    grid = (B, nblk)

    idxr, wr = pl.pallas_call(
        _knn_kernel,
        grid=grid,
        in_specs=[
            pl.BlockSpec((1, 3, NB), lambda b, i: (b, 0, i)),
            pl.BlockSpec((1, 3, M), lambda b, i: (b, 0, 0)),
        ],
        out_specs=[
            pl.BlockSpec((1, 1, 3 * NB), lambda b, i: (b * nblk + i, 0, 0)),
            pl.BlockSpec((1, 1, 3 * NB), lambda b, i: (b * nblk + i, 0, 0)),
        ],
        out_shape=[
            jax.ShapeDtypeStruct((B * nblk, 1, 3 * NB), jnp.int32),
            jax.ShapeDtypeStruct((B * nblk, 1, 3 * NB), jnp.float32),
        ],
    )(x1t, x2t)

    mesh = plsc.VectorSubcoreMesh(core_axis_name="c", subcore_axis_name="s")
    interp, h0d, h1d = pl.kernel(
        _interp_sc_kernel,
        mesh=mesh,
        compiler_params=pltpu.CompilerParams(use_tc_tiling_on_sc=False),
        out_type=[
            jax.ShapeDtypeStruct((B * N, C2), jnp.float32),
            jax.ShapeDtypeStruct((B, N, CH0), jnp.float32),
            jax.ShapeDtypeStruct((B, N, CH1), jnp.float32),
        ],
        scratch_types=[
            pltpu.VMEM((3 * NB,), jnp.int32),
            pltpu.VMEM((3 * NB,), jnp.int32),
            pltpu.VMEM((3 * NB, 16), jnp.float32),
            pltpu.VMEM((3 * NB, 16), jnp.float32),
            pltpu.VMEM((CP, C2), jnp.float32),
            pltpu.VMEM((CP, C2), jnp.float32),
            pltpu.VMEM((CP, C2), jnp.float32),
            pltpu.VMEM((CP, C2), jnp.float32),
            pltpu.VMEM((CP, C2), jnp.float32),
            pltpu.VMEM((CP, C2), jnp.float32),
            pltpu.VMEM((CP, C2), jnp.float32),
            pltpu.SemaphoreType.DMA,
            pltpu.SemaphoreType.DMA,
            pltpu.SemaphoreType.DMA,
        ],
    )(idxr.reshape(-1),
      jnp.broadcast_to(wr.reshape(-1)[:, None], (B * N * 3, 16)),
      points2.reshape(B * M, C2))

    def _on0(p, v, last):
        return jnp.where(p == 0, v, last)

    def _on1(p, v, last):
        return jnp.where(p == 1, v, last)

    def _on2(p, v, last):
        return jnp.where(p == 2, v, last)

    _, _, out = pl.pallas_call(
        functools.partial(_mlp_fused_kernel, count_inv),
        grid=(3, B, nblk),
        in_specs=[
            pl.BlockSpec((1, NB, C1),
                         lambda p, b, i: (_on0(p, b, B - 1),
                                          _on0(p, i, nblk - 1), 0)),
            pl.BlockSpec((1, NB, C2),
                         lambda p, b, i: (_on0(p, b, B - 1),
                                          _on0(p, i, nblk - 1), 0)),
            pl.BlockSpec((CH0, C1 + C2), lambda p, b, i: (0, 0)),
            pl.BlockSpec((1, CH0), lambda p, b, i: (0, 0)),
            pl.BlockSpec((1, CH0), lambda p, b, i: (0, 0)),
            pl.BlockSpec((1, CH0), lambda p, b, i: (0, 0)),
            pl.BlockSpec((CH1, CH0), lambda p, b, i: (0, 0)),
            pl.BlockSpec((1, CH1), lambda p, b, i: (0, 0)),
            pl.BlockSpec((1, CH1), lambda p, b, i: (0, 0)),
            pl.BlockSpec((1, CH1), lambda p, b, i: (0, 0)),
            pl.BlockSpec((1, NB, CH0),
                         lambda p, b, i: (_on1(p, b, B - 1),
                                          _on1(p, i, nblk - 1), 0)),
            pl.BlockSpec((1, NB, CH1),
                         lambda p, b, i: (_on2(p, b, B - 1),
                                          _on2(p, i, nblk - 1), 0)),
        ],
        out_specs=[
            pl.BlockSpec((1, NB, CH0),
                         lambda p, b, i: (_on0(p, b, 0), _on0(p, i, 0), 0)),
            pl.BlockSpec((1, NB, CH1),
                         lambda p, b, i: (_on1(p, b, 0), _on1(p, i, 0), 0)),
            pl.BlockSpec((1, NB, CH1),
                         lambda p, b, i: (_on2(p, b, 0), _on2(p, i, 0), 0)),
        ],
        out_shape=[
            jax.ShapeDtypeStruct((B, N, CH0), jnp.float32),
            jax.ShapeDtypeStruct((B, N, CH1), jnp.float32),
            jax.ShapeDtypeStruct((B, N, CH1), jnp.float32),
        ],
        scratch_shapes=[
            pltpu.VMEM((1, CH0), jnp.float32),
            pltpu.VMEM((1, CH0), jnp.float32),
            pltpu.VMEM((1, CH1), jnp.float32),
            pltpu.VMEM((1, CH1), jnp.float32),
        ],
        input_output_aliases={10: 0, 11: 1},
    )(points1, interp.reshape(B, N, C2), W0, b0.reshape(1, -1),
      g0.reshape(1, -1), be0.reshape(1, -1), W1, b1.reshape(1, -1),
      g1.reshape(1, -1), be1.reshape(1, -1), h0d, h1d)

    return out


# speedup vs baseline: 1.0805x; 1.0805x over previous
import functools
import jax
import jax.numpy as jnp
from jax import lax
from jax.experimental import pallas as pl
from jax.experimental.pallas import tpu as pltpu
from jax.experimental.pallas import tpu_sc as plsc

NB = 256
CP = 32


def _knn_kernel(x1t_ref, x2t_ref, idx_ref, w_ref):
    b = pl.program_id(0)
    x1 = x1t_ref[0]
    x2 = x2t_ref[0]
    M = x2.shape[1]
    cross = jax.lax.dot_general(x2, x1, (((0,), (0,)), ((), ())),
                                preferred_element_type=jnp.float32)
    n1 = jnp.sum(x1 * x1, axis=0)[None, :]
    n2 = jnp.sum(x2 * x2, axis=0)[:, None]
    d = (-2.0 * cross + n1) + n2
    iota0 = jax.lax.broadcasted_iota(jnp.int32, d.shape, 0)

    firsts = []
    ws = []
    for _ in range(3):
        mval = jnp.min(d, axis=0, keepdims=True)
        first = jnp.min(jnp.where(d == mval, iota0, M), axis=0,
                        keepdims=True)
        ws.append(1.0 / (mval + 1e-8))
        firsts.append(first + b * M)
        sel = iota0 == first
        d = jnp.where(sel, jnp.inf, d)
    norm = ws[0] + ws[1] + ws[2]
    idx_ref[0] = jnp.concatenate(firsts, axis=1)
    w_ref[0] = jnp.concatenate([w / norm for w in ws], axis=1)


def _interp_sc_kernel(idx_hbm, w_hbm, table_hbm, out_hbm,
                      ija_v, ijb_v, wja_v, wjb_v,
                      ra0_v, ra1_v, ra2_v, rb0_v, rb1_v, rb2_v,
                      out_v, sema, semb, semj):
    info = plsc.get_sparse_core_info()
    NC, NS, L = info.num_cores, info.num_subcores, info.num_lanes
    NW = NC * NS
    wid = lax.axis_index("s") * NC + lax.axis_index("c")
    total = out_hbm.shape[0]
    per_tile = total // NW
    n_chunks = per_tile // CP
    cpj = NB // CP
    n_jblk = per_tile // NB
    tile_base = wid * per_tile
    rbufs = ((ra0_v, ra1_v, ra2_v, sema), (rb0_v, rb1_v, rb2_v, semb))
    jbufs = (ija_v, wja_v), (ijb_v, wjb_v)

    def start_jblk(jb, jp):
        iv, wv = jbufs[jp]
        j = (tile_base + jb * NB) // NB
        base = pl.multiple_of(j * (3 * NB), 64)
        pltpu.async_copy(idx_hbm.at[pl.ds(base, 3 * NB)], iv, semj)
        pltpu.async_copy(w_hbm.at[pl.ds(base, 3 * NB), :], wv, semj)

    def start_chunk(off, rp, jpo):
        iv = jbufs[jpo][0]
        rows = rbufs[rp]
        for k in range(3):
            pltpu.async_copy(
                table_hbm.at[iv.at[pl.ds(k * NB + off, CP)]], rows[k],
                rows[3])

    def finish_chunk(jb, off, rp, jpo):
        iv, wv = jbufs[jpo]
        p0 = tile_base + jb * NB + off
        rows = rbufs[rp]
        for k in range(3):
            pltpu.make_async_copy(
                table_hbm.at[iv.at[pl.ds(k * NB + off, CP)]], rows[k],
                rows[3]).wait()

        def point_body(pt, _):
            v0 = wv[off + pt, :]
            v1 = wv[NB + off + pt, :]
            v2 = wv[2 * NB + off + pt, :]
            for s in range(out_v.shape[1] // L):
                sl = pl.ds(s * L, L)
                out_v[pt, sl] = (v0 * rows[0][pt, sl] + v1 * rows[1][pt, sl]
                                 + v2 * rows[2][pt, sl])
            return 0

        lax.fori_loop(0, CP, point_body, 0)
        pltpu.sync_copy(out_v, out_hbm.at[pl.ds(pl.multiple_of(p0, CP), CP)])

    def wait_jblk(jp):
        iv, wv = jbufs[jp]
        pltpu.make_async_copy(idx_hbm.at[pl.ds(0, 3 * NB)], iv, semj).wait()
        pltpu.make_async_copy(w_hbm.at[pl.ds(0, 3 * NB), :], wv, semj).wait()

    start_jblk(0, 0)
    wait_jblk(0)
    start_chunk(0, 0, 0)

    def jpair_body(t, _):
        for half in range(2):
            jb = 2 * t + half
            jp = half

            @pl.when(jb + 1 < n_jblk)
            def _():
                start_jblk(jb + 1, jp ^ 1)

            for cc in range(cpj):
                if cc < cpj - 1:
                    start_chunk((cc + 1) * CP, (cc + 1) % 2, jp)
                else:
                    @pl.when(jb + 1 < n_jblk)
                    def _():
                        wait_jblk(jp ^ 1)
                        start_chunk(0, 0, jp ^ 1)

                finish_chunk(jb, cc * CP, cc % 2, jp)
        return 0

    lax.fori_loop(0, n_jblk // 2, jpair_body, 0)


def _mlp1_kernel(p1_ref, it_ref, w0_ref, b0_ref, h0_ref, s_ref, q_ref):
    b = pl.program_id(0)
    i = pl.program_id(1)
    f = jnp.concatenate([p1_ref[0], it_ref[0]], axis=1)
    h = jax.lax.dot_general(f, w0_ref[...], (((1,), (1,)), ((), ())),
                            preferred_element_type=jnp.float32)
    h = h + b0_ref[...]
    h0_ref[0] = h

    @pl.when(jnp.logical_and(b == 0, i == 0))
    def _():
        s_ref[...] = jnp.zeros_like(s_ref)
        q_ref[...] = jnp.zeros_like(q_ref)

    s_ref[...] += jnp.sum(h, axis=0, keepdims=True)
    q_ref[...] += jnp.sum(h * h, axis=0, keepdims=True)


def _mlp2_kernel(count_inv, h0_ref, s_ref, q_ref, g_ref, be_ref, w1_ref,
                 b1_ref, h1_ref, s2_ref, q2_ref):
    b = pl.program_id(0)
    i = pl.program_id(1)
    mean = s_ref[...] * count_inv
    var = q_ref[...] * count_inv - mean * mean
    inv = jax.lax.rsqrt(var + 1e-5)
    scale = g_ref[...] * inv
    shift = be_ref[...] - mean * scale
    y = jnp.maximum(h0_ref[0] * scale + shift, 0.0)
    h = jax.lax.dot_general(y, w1_ref[...], (((1,), (1,)), ((), ())),
                            preferred_element_type=jnp.float32)
    h = h + b1_ref[...]
    h1_ref[0] = h

    @pl.when(jnp.logical_and(b == 0, i == 0))
    def _():
        s2_ref[...] = jnp.zeros_like(s2_ref)
        q2_ref[...] = jnp.zeros_like(q2_ref)

    s2_ref[...] += jnp.sum(h, axis=0, keepdims=True)
    q2_ref[...] += jnp.sum(h * h, axis=0, keepdims=True)


def _bn_out_kernel(count_inv, h1_ref, s_ref, q_ref, g_ref, be_ref, out_ref):
    mean = s_ref[...] * count_inv
    var = q_ref[...] * count_inv - mean * mean
    inv = jax.lax.rsqrt(var + 1e-5)
    scale = g_ref[...] * inv
    shift = be_ref[...] - mean * scale
    out_ref[0] = jnp.maximum(h1_ref[0] * scale + shift, 0.0)


@jax.jit
def kernel(xyz1, xyz2, points1, points2, W0, b0, g0, be0, W1, b1, g1, be1):
    B, N, _ = xyz1.shape
    M = xyz2.shape[1]
    C1 = points1.shape[-1]
    C2 = points2.shape[-1]
    CH0 = W0.shape[0]
    CH1 = W1.shape[0]
    x1t = jnp.transpose(xyz1, (0, 2, 1))
    x2t = jnp.transpose(xyz2, (0, 2, 1))
    count_inv = 1.0 / float(B * N)
    nblk = N // NB
    grid = (B, nblk)

    idxr, wr = pl.pallas_call(
        _knn_kernel,
        grid=grid,
        in_specs=[
            pl.BlockSpec((1, 3, NB), lambda b, i: (b, 0, i)),
            pl.BlockSpec((1, 3, M), lambda b, i: (b, 0, 0)),
        ],
        out_specs=[
            pl.BlockSpec((1, 1, 3 * NB), lambda b, i: (b * nblk + i, 0, 0)),
            pl.BlockSpec((1, 1, 3 * NB), lambda b, i: (b * nblk + i, 0, 0)),
        ],
        out_shape=[
            jax.ShapeDtypeStruct((B * nblk, 1, 3 * NB), jnp.int32),
            jax.ShapeDtypeStruct((B * nblk, 1, 3 * NB), jnp.float32),
        ],
    )(x1t, x2t)

    mesh = plsc.VectorSubcoreMesh(core_axis_name="c", subcore_axis_name="s")
    interp = pl.kernel(
        _interp_sc_kernel,
        mesh=mesh,
        compiler_params=pltpu.CompilerParams(use_tc_tiling_on_sc=False),
        out_type=jax.ShapeDtypeStruct((B * N, C2), jnp.float32),
        scratch_types=[
            pltpu.VMEM((3 * NB,), jnp.int32),
            pltpu.VMEM((3 * NB,), jnp.int32),
            pltpu.VMEM((3 * NB, 16), jnp.float32),
            pltpu.VMEM((3 * NB, 16), jnp.float32),
            pltpu.VMEM((CP, C2), jnp.float32),
            pltpu.VMEM((CP, C2), jnp.float32),
            pltpu.VMEM((CP, C2), jnp.float32),
            pltpu.VMEM((CP, C2), jnp.float32),
            pltpu.VMEM((CP, C2), jnp.float32),
            pltpu.VMEM((CP, C2), jnp.float32),
            pltpu.VMEM((CP, C2), jnp.float32),
            pltpu.SemaphoreType.DMA,
            pltpu.SemaphoreType.DMA,
            pltpu.SemaphoreType.DMA,
        ],
    )(idxr.reshape(-1),
      jnp.broadcast_to(wr.reshape(-1)[:, None], (B * N * 3, 16)),
      points2.reshape(B * M, C2))

    h0, s0, q0 = pl.pallas_call(
        _mlp1_kernel,
        grid=grid,
        in_specs=[
            pl.BlockSpec((1, NB, C1), lambda b, i: (b, i, 0)),
            pl.BlockSpec((1, NB, C2), lambda b, i: (b, i, 0)),
            pl.BlockSpec((CH0, C1 + C2), lambda b, i: (0, 0)),
            pl.BlockSpec((1, CH0), lambda b, i: (0, 0)),
        ],
        out_specs=[
            pl.BlockSpec((1, NB, CH0), lambda b, i: (b, i, 0)),
            pl.BlockSpec((1, CH0), lambda b, i: (0, 0)),
            pl.BlockSpec((1, CH0), lambda b, i: (0, 0)),
        ],
        out_shape=[
            jax.ShapeDtypeStruct((B, N, CH0), jnp.float32),
            jax.ShapeDtypeStruct((1, CH0), jnp.float32),
            jax.ShapeDtypeStruct((1, CH0), jnp.float32),
        ],
    )(points1, interp.reshape(B, N, C2), W0, b0.reshape(1, -1))

    h1, s1, q1 = pl.pallas_call(
        functools.partial(_mlp2_kernel, count_inv),
        grid=grid,
        in_specs=[
            pl.BlockSpec((1, NB, CH0), lambda b, i: (b, i, 0)),
            pl.BlockSpec((1, CH0), lambda b, i: (0, 0)),
            pl.BlockSpec((1, CH0), lambda b, i: (0, 0)),
            pl.BlockSpec((1, CH0), lambda b, i: (0, 0)),
            pl.BlockSpec((1, CH0), lambda b, i: (0, 0)),
            pl.BlockSpec((CH1, CH0), lambda b, i: (0, 0)),
            pl.BlockSpec((1, CH1), lambda b, i: (0, 0)),
        ],
        out_specs=[
            pl.BlockSpec((1, NB, CH1), lambda b, i: (b, i, 0)),
            pl.BlockSpec((1, CH1), lambda b, i: (0, 0)),
            pl.BlockSpec((1, CH1), lambda b, i: (0, 0)),
        ],
        out_shape=[
            jax.ShapeDtypeStruct((B, N, CH1), jnp.float32),
            jax.ShapeDtypeStruct((1, CH1), jnp.float32),
            jax.ShapeDtypeStruct((1, CH1), jnp.float32),
        ],
    )(h0, s0, q0, g0.reshape(1, -1), be0.reshape(1, -1), W1,
      b1.reshape(1, -1))

    out = pl.pallas_call(
        functools.partial(_bn_out_kernel, count_inv),
        grid=grid,
        in_specs=[
            pl.BlockSpec((1, NB, CH1), lambda b, i: (b, i, 0)),
            pl.BlockSpec((1, CH1), lambda b, i: (0, 0)),
            pl.BlockSpec((1, CH1), lambda b, i: (0, 0)),
            pl.BlockSpec((1, CH1), lambda b, i: (0, 0)),
            pl.BlockSpec((1, CH1), lambda b, i: (0, 0)),
        ],
        out_specs=pl.BlockSpec((1, NB, CH1), lambda b, i: (b, i, 0)),
        out_shape=jax.ShapeDtypeStruct((B, N, CH1), jnp.float32),
    )(h1, s1, q1, g1.reshape(1, -1), be1.reshape(1, -1))

    return out


# speedup vs baseline: 1.4772x; 1.3672x over previous
import functools
import jax
import jax.numpy as jnp
from jax import lax
from jax.experimental import pallas as pl
from jax.experimental.pallas import tpu as pltpu
from jax.experimental.pallas import tpu_sc as plsc

NB = 256
NBM = 1024
CP = 32


def _knn_kernel(x1t_ref, x2t_ref, idx_ref, w_ref):
    b = pl.program_id(0)
    x1 = x1t_ref[0]
    x2 = x2t_ref[0]
    M = x2.shape[1]
    cross = jax.lax.dot_general(x2, x1, (((0,), (0,)), ((), ())),
                                preferred_element_type=jnp.float32)
    n1 = jnp.sum(x1 * x1, axis=0)[None, :]
    n2 = jnp.sum(x2 * x2, axis=0)[:, None]
    d = (-2.0 * cross + n1) + n2
    iota0 = jax.lax.broadcasted_iota(jnp.int32, d.shape, 0)

    firsts = []
    ws = []
    for _ in range(3):
        mval = jnp.min(d, axis=0, keepdims=True)
        first = jnp.min(jnp.where(d == mval, iota0, M), axis=0,
                        keepdims=True)
        ws.append(1.0 / (mval + 1e-8))
        firsts.append(first + b * M)
        sel = iota0 == first
        d = jnp.where(sel, jnp.inf, d)
    norm = ws[0] + ws[1] + ws[2]
    idx_ref[0] = jnp.concatenate(firsts, axis=1)
    w_ref[0] = jnp.concatenate([w / norm for w in ws], axis=1)


def _interp_sc_kernel(idx_hbm, w_hbm, table_hbm, out_hbm,
                      ija_v, ijb_v, wja_v, wjb_v,
                      ra0_v, ra1_v, ra2_v, rb0_v, rb1_v, rb2_v,
                      out_v, sema, semb, semj):
    info = plsc.get_sparse_core_info()
    NC, NS, L = info.num_cores, info.num_subcores, info.num_lanes
    NW = NC * NS
    wid = lax.axis_index("s") * NC + lax.axis_index("c")
    total = out_hbm.shape[0]
    per_tile = total // NW
    n_chunks = per_tile // CP
    cpj = NB // CP
    n_jblk = per_tile // NB
    tile_base = wid * per_tile
    rbufs = ((ra0_v, ra1_v, ra2_v, sema), (rb0_v, rb1_v, rb2_v, semb))
    jbufs = (ija_v, wja_v), (ijb_v, wjb_v)

    def start_jblk(jb, jp):
        iv, wv = jbufs[jp]
        j = (tile_base + jb * NB) // NB
        base = pl.multiple_of(j * (3 * NB), 64)
        pltpu.async_copy(idx_hbm.at[pl.ds(base, 3 * NB)], iv, semj)
        pltpu.async_copy(w_hbm.at[pl.ds(base, 3 * NB), :], wv, semj)

    def start_chunk(off, rp, jpo):
        iv = jbufs[jpo][0]
        rows = rbufs[rp]
        for k in range(3):
            pltpu.async_copy(
                table_hbm.at[iv.at[pl.ds(k * NB + off, CP)]], rows[k],
                rows[3])

    def finish_chunk(jb, off, rp, jpo):
        iv, wv = jbufs[jpo]
        p0 = tile_base + jb * NB + off
        rows = rbufs[rp]
        for k in range(3):
            pltpu.make_async_copy(
                table_hbm.at[iv.at[pl.ds(k * NB + off, CP)]], rows[k],
                rows[3]).wait()

        def point_body(pt, _):
            v0 = wv[off + pt, :]
            v1 = wv[NB + off + pt, :]
            v2 = wv[2 * NB + off + pt, :]
            for s in range(out_v.shape[1] // L):
                sl = pl.ds(s * L, L)
                out_v[pt, sl] = (v0 * rows[0][pt, sl] + v1 * rows[1][pt, sl]
                                 + v2 * rows[2][pt, sl])
            return 0

        lax.fori_loop(0, CP, point_body, 0)
        pltpu.sync_copy(out_v, out_hbm.at[pl.ds(pl.multiple_of(p0, CP), CP)])

    def wait_jblk(jp):
        iv, wv = jbufs[jp]
        pltpu.make_async_copy(idx_hbm.at[pl.ds(0, 3 * NB)], iv, semj).wait()
        pltpu.make_async_copy(w_hbm.at[pl.ds(0, 3 * NB), :], wv, semj).wait()

    start_jblk(0, 0)
    wait_jblk(0)
    start_chunk(0, 0, 0)

    def jpair_body(t, _):
        for half in range(2):
            jb = 2 * t + half
            jp = half

            @pl.when(jb + 1 < n_jblk)
            def _():
                start_jblk(jb + 1, jp ^ 1)

            for cc in range(cpj):
                if cc < cpj - 1:
                    start_chunk((cc + 1) * CP, (cc + 1) % 2, jp)
                else:
                    @pl.when(jb + 1 < n_jblk)
                    def _():
                        wait_jblk(jp ^ 1)
                        start_chunk(0, 0, jp ^ 1)

                finish_chunk(jb, cc * CP, cc % 2, jp)
        return 0

    lax.fori_loop(0, n_jblk // 2, jpair_body, 0)


def _mlp1_kernel(p1_ref, it_ref, w0_ref, b0_ref, h0_ref, s_ref, q_ref):
    b = pl.program_id(0)
    i = pl.program_id(1)
    f = jnp.concatenate([p1_ref[0], it_ref[0]], axis=1)
    h = jax.lax.dot_general(f, w0_ref[...], (((1,), (1,)), ((), ())),
                            preferred_element_type=jnp.float32)
    h = h + b0_ref[...]
    h0_ref[0] = h

    @pl.when(jnp.logical_and(b == 0, i == 0))
    def _():
        s_ref[...] = jnp.zeros_like(s_ref)
        q_ref[...] = jnp.zeros_like(q_ref)

    s_ref[...] += jnp.sum(h, axis=0, keepdims=True)
    q_ref[...] += jnp.sum(h * h, axis=0, keepdims=True)


def _mlp2_kernel(count_inv, h0_ref, s_ref, q_ref, g_ref, be_ref, w1_ref,
                 b1_ref, h1_ref, s2_ref, q2_ref):
    b = pl.program_id(0)
    i = pl.program_id(1)
    mean = s_ref[...] * count_inv
    var = q_ref[...] * count_inv - mean * mean
    inv = jax.lax.rsqrt(var + 1e-5)
    scale = g_ref[...] * inv
    shift = be_ref[...] - mean * scale
    y = jnp.maximum(h0_ref[0] * scale + shift, 0.0)
    h = jax.lax.dot_general(y, w1_ref[...], (((1,), (1,)), ((), ())),
                            preferred_element_type=jnp.float32)
    h = h + b1_ref[...]
    h1_ref[0] = h

    @pl.when(jnp.logical_and(b == 0, i == 0))
    def _():
        s2_ref[...] = jnp.zeros_like(s2_ref)
        q2_ref[...] = jnp.zeros_like(q2_ref)

    s2_ref[...] += jnp.sum(h, axis=0, keepdims=True)
    q2_ref[...] += jnp.sum(h * h, axis=0, keepdims=True)


def _bn_out_kernel(count_inv, h1_ref, s_ref, q_ref, g_ref, be_ref, out_ref):
    mean = s_ref[...] * count_inv
    var = q_ref[...] * count_inv - mean * mean
    inv = jax.lax.rsqrt(var + 1e-5)
    scale = g_ref[...] * inv
    shift = be_ref[...] - mean * scale
    out_ref[0] = jnp.maximum(h1_ref[0] * scale + shift, 0.0)


@jax.jit
def kernel(xyz1, xyz2, points1, points2, W0, b0, g0, be0, W1, b1, g1, be1):
    B, N, _ = xyz1.shape
    M = xyz2.shape[1]
    C1 = points1.shape[-1]
    C2 = points2.shape[-1]
    CH0 = W0.shape[0]
    CH1 = W1.shape[0]
    x1t = jnp.transpose(xyz1, (0, 2, 1))
    x2t = jnp.transpose(xyz2, (0, 2, 1))
    count_inv = 1.0 / float(B * N)
    nblk = N // NB
    grid = (B, nblk)
    gridm = (B, N // NBM)

    idxr, wr = pl.pallas_call(
        _knn_kernel,
        grid=grid,
        in_specs=[
            pl.BlockSpec((1, 3, NB), lambda b, i: (b, 0, i)),
            pl.BlockSpec((1, 3, M), lambda b, i: (b, 0, 0)),
        ],
        out_specs=[
            pl.BlockSpec((1, 1, 3 * NB), lambda b, i: (b * nblk + i, 0, 0)),
            pl.BlockSpec((1, 1, 3 * NB), lambda b, i: (b * nblk + i, 0, 0)),
        ],
        out_shape=[
            jax.ShapeDtypeStruct((B * nblk, 1, 3 * NB), jnp.int32),
            jax.ShapeDtypeStruct((B * nblk, 1, 3 * NB), jnp.float32),
        ],
    )(x1t, x2t)

    mesh = plsc.VectorSubcoreMesh(core_axis_name="c", subcore_axis_name="s")
    interp = pl.kernel(
        _interp_sc_kernel,
        mesh=mesh,
        compiler_params=pltpu.CompilerParams(use_tc_tiling_on_sc=False),
        out_type=jax.ShapeDtypeStruct((B * N, C2), jnp.float32),
        scratch_types=[
            pltpu.VMEM((3 * NB,), jnp.int32),
            pltpu.VMEM((3 * NB,), jnp.int32),
            pltpu.VMEM((3 * NB, 16), jnp.float32),
            pltpu.VMEM((3 * NB, 16), jnp.float32),
            pltpu.VMEM((CP, C2), jnp.float32),
            pltpu.VMEM((CP, C2), jnp.float32),
            pltpu.VMEM((CP, C2), jnp.float32),
            pltpu.VMEM((CP, C2), jnp.float32),
            pltpu.VMEM((CP, C2), jnp.float32),
            pltpu.VMEM((CP, C2), jnp.float32),
            pltpu.VMEM((CP, C2), jnp.float32),
            pltpu.SemaphoreType.DMA,
            pltpu.SemaphoreType.DMA,
            pltpu.SemaphoreType.DMA,
        ],
    )(idxr.reshape(-1),
      jnp.broadcast_to(wr.reshape(-1)[:, None], (B * N * 3, 16)),
      points2.reshape(B * M, C2))

    h0, s0, q0 = pl.pallas_call(
        _mlp1_kernel,
        grid=gridm,
        in_specs=[
            pl.BlockSpec((1, NBM, C1), lambda b, i: (b, i, 0)),
            pl.BlockSpec((1, NBM, C2), lambda b, i: (b, i, 0)),
            pl.BlockSpec((CH0, C1 + C2), lambda b, i: (0, 0)),
            pl.BlockSpec((1, CH0), lambda b, i: (0, 0)),
        ],
        out_specs=[
            pl.BlockSpec((1, NBM, CH0), lambda b, i: (b, i, 0)),
            pl.BlockSpec((1, CH0), lambda b, i: (0, 0)),
            pl.BlockSpec((1, CH0), lambda b, i: (0, 0)),
        ],
        out_shape=[
            jax.ShapeDtypeStruct((B, N, CH0), jnp.float32),
            jax.ShapeDtypeStruct((1, CH0), jnp.float32),
            jax.ShapeDtypeStruct((1, CH0), jnp.float32),
        ],
    )(points1, interp.reshape(B, N, C2), W0, b0.reshape(1, -1))

    h1, s1, q1 = pl.pallas_call(
        functools.partial(_mlp2_kernel, count_inv),
        grid=gridm,
        in_specs=[
            pl.BlockSpec((1, NBM, CH0), lambda b, i: (b, i, 0)),
            pl.BlockSpec((1, CH0), lambda b, i: (0, 0)),
            pl.BlockSpec((1, CH0), lambda b, i: (0, 0)),
            pl.BlockSpec((1, CH0), lambda b, i: (0, 0)),
            pl.BlockSpec((1, CH0), lambda b, i: (0, 0)),
            pl.BlockSpec((CH1, CH0), lambda b, i: (0, 0)),
            pl.BlockSpec((1, CH1), lambda b, i: (0, 0)),
        ],
        out_specs=[
            pl.BlockSpec((1, NBM, CH1), lambda b, i: (b, i, 0)),
            pl.BlockSpec((1, CH1), lambda b, i: (0, 0)),
            pl.BlockSpec((1, CH1), lambda b, i: (0, 0)),
        ],
        out_shape=[
            jax.ShapeDtypeStruct((B, N, CH1), jnp.float32),
            jax.ShapeDtypeStruct((1, CH1), jnp.float32),
            jax.ShapeDtypeStruct((1, CH1), jnp.float32),
        ],
    )(h0, s0, q0, g0.reshape(1, -1), be0.reshape(1, -1), W1,
      b1.reshape(1, -1))

    out = pl.pallas_call(
        functools.partial(_bn_out_kernel, count_inv),
        grid=gridm,
        in_specs=[
            pl.BlockSpec((1, NBM, CH1), lambda b, i: (b, i, 0)),
            pl.BlockSpec((1, CH1), lambda b, i: (0, 0)),
            pl.BlockSpec((1, CH1), lambda b, i: (0, 0)),
            pl.BlockSpec((1, CH1), lambda b, i: (0, 0)),
            pl.BlockSpec((1, CH1), lambda b, i: (0, 0)),
        ],
        out_specs=pl.BlockSpec((1, NBM, CH1), lambda b, i: (b, i, 0)),
        out_shape=jax.ShapeDtypeStruct((B, N, CH1), jnp.float32),
    )(h1, s1, q1, g1.reshape(1, -1), be1.reshape(1, -1))

    return out
